# Initial kernel scaffold; baseline (speedup 1.0000x reference)
import jax
import jax.numpy as jnp
from jax.experimental import pallas as pl


def kernel(x, xyz, knn0, knn1, ids1, back_nn1, params):
    raise NotImplementedError("write your pallas kernel here")


# speedup vs baseline: 1.8710x; 1.8710x over previous
import functools

import jax
import jax.numpy as jnp
from jax import lax
from jax.experimental import pallas as pl
from jax.experimental.pallas import tpu as pltpu
from jax.experimental.pallas import tpu_sc as plsc

N0, N1, KNN = 50000, 12500, 16
N0P = 51200
N1P = 12800
N0BP = 51200
_S = float((1.0 + 1e-5) ** -0.5)
_NW = 32
D = 128


def _gelu(v):
    return 0.5 * v * (1.0 + lax.erf(v * 0.7071067811865476))


def _sc_gather_fn(V, B, C):
    b_per_w = B // _NW
    nch = b_per_w // C
    mesh = plsc.VectorSubcoreMesh(core_axis_name="c", subcore_axis_name="s")

    @functools.partial(
        pl.kernel,
        out_type=jax.ShapeDtypeStruct((B, D), jnp.float32),
        mesh=mesh,
        scratch_types=[
            pltpu.VMEM((C,), jnp.int32),
            pltpu.VMEM((C, D), jnp.float32),
            pltpu.SemaphoreType.DMA,
        ],
    )
    def gather_k(table_hbm, idx_hbm, out_hbm, idx_v, rows_v, sem):
        wid = lax.axis_index("s") * 2 + lax.axis_index("c")
        wbase = wid * b_per_w

        def body(c, carry):
            base = wbase + c * C
            pltpu.sync_copy(idx_hbm.at[pl.ds(base, C)], idx_v)
            pltpu.async_copy(table_hbm.at[idx_v], rows_v, sem).wait()
            pltpu.sync_copy(rows_v, out_hbm.at[pl.ds(base, C)])
            return carry

        lax.fori_loop(0, nch, body, 0)

    return gather_k


def _gather_rows(table, idx, chunk):
    V, d = table.shape
    B = idx.shape[0]
    assert d == D and B % (_NW * chunk) == 0 and chunk % 8 == 0, (V, d, B, chunk)
    return _sc_gather_fn(V, B, chunk)(table, idx)


def _row_bs(R, d):
    return pl.BlockSpec((R, d), lambda i: (i, 0))


def _full_bs(shape):
    nd = len(shape)
    return pl.BlockSpec(shape, lambda i: (0,) * nd)


def _edge_bs(R):
    return pl.BlockSpec((KNN, R, D), lambda i: (0, i, 0))


def _dot(a, b):
    return jnp.dot(a, b, preferred_element_type=jnp.float32)


def _tc_matmul(x, w, R):
    N, Di = x.shape
    Do = w.shape[1]

    def body(x_ref, w_ref, o_ref):
        o_ref[...] = _dot(x_ref[...], w_ref[...])

    return pl.pallas_call(
        body,
        grid=(N // R,),
        in_specs=[_row_bs(R, Di), _full_bs(w.shape)],
        out_specs=_row_bs(R, Do),
        out_shape=jax.ShapeDtypeStruct((N, Do), jnp.float32),
    )(x, w)


def _tc_embed3(x, w1, w2, w3, add, R):
    N, Di = x.shape
    Do = w3.shape[1]

    def body(x_ref, w1_ref, w2_ref, w3_ref, a_ref, o_ref):
        h = _gelu(_dot(x_ref[...], w1_ref[...]))
        h = _gelu(_dot(h, w2_ref[...]))
        o_ref[...] = _dot(h, w3_ref[...]) + a_ref[...]

    return pl.pallas_call(
        body,
        grid=(N // R,),
        in_specs=[_row_bs(R, Di), _full_bs(w1.shape), _full_bs(w2.shape),
                  _full_bs(w3.shape), _row_bs(R, Do)],
        out_specs=_row_bs(R, Do),
        out_shape=jax.ShapeDtypeStruct((N, Do), jnp.float32),
    )(x, w1, w2, w3, add)


def _tc_mlp_res(x, w1, b1, w2s, R):
    N, d = x.shape

    def body(x_ref, w1_ref, b1_ref, w2_ref, o_ref):
        h = _gelu(_dot(x_ref[...], w1_ref[...]) + b1_ref[...])
        o_ref[...] = x_ref[...] + _dot(h, w2_ref[...])

    return pl.pallas_call(
        body,
        grid=(N // R,),
        in_specs=[_row_bs(R, d), _full_bs(w1.shape), _full_bs((1,) + b1.shape),
                  _full_bs(w2s.shape)],
        out_specs=_row_bs(R, d),
        out_shape=jax.ShapeDtypeStruct((N, d), jnp.float32),
    )(x, w1, b1[None, :], w2s)


def _tc_lfp_max(g3, y, res, svec, R):
    N = y.shape[0]

    def body(g_ref, y_ref, r_ref, s_ref, o_ref):
        acc = g_ref[0]
        for k in range(1, KNN):
            acc = jnp.maximum(acc, g_ref[k])
        o_ref[...] = r_ref[...] + (acc - y_ref[...]) * s_ref[...]

    return pl.pallas_call(
        body,
        grid=(N // R,),
        in_specs=[_edge_bs(R), _row_bs(R, D), _row_bs(R, D), _full_bs((1, D))],
        out_specs=_row_bs(R, D),
        out_shape=jax.ShapeDtypeStruct((N, D), jnp.float32),
    )(g3, y, res, svec[None, :])


def _tc_edge0(tg, center, w1, w2, w3, esel, svec, N, R):
    def body(tg_ref, c_ref, w1_ref, w2_ref, w3_ref, e_ref, s_ref, o_ref):
        c = c_ref[...]
        acc = None
        for k in range(KNN):
            e = tg_ref[k] - c
            h = _gelu(_dot(e, w1_ref[...]))
            h = _gelu(_dot(h, w2_ref[...]))
            h = _dot(h, w3_ref[...]) + _dot(e, e_ref[...])
            acc = h if acc is None else jnp.maximum(acc, h)
        o_ref[...] = acc * s_ref[...]

    return pl.pallas_call(
        body,
        grid=(N // R,),
        in_specs=[_edge_bs(R), _row_bs(R, D), _full_bs(w1.shape), _full_bs(w2.shape),
                  _full_bs(w3.shape), _full_bs(esel.shape), _full_bs((1, D))],
        out_specs=_row_bs(R, D),
        out_shape=jax.ShapeDtypeStruct((N, D), jnp.float32),
    )(tg, center, w1, w2, w3, esel, svec[None, :])


def _tc_edge1(tg, center, w1, w2, w3, esel, wproj, res, N, R):
    def body(tg_ref, c_ref, w1_ref, w2_ref, w3_ref, e_ref, wp_ref, r_ref, o_ref):
        c = c_ref[...]
        acc = None
        for k in range(KNN):
            e = tg_ref[k] - c
            h = _gelu(_dot(e, w1_ref[...]))
            h = _gelu(_dot(h, w2_ref[...]))
            h = _dot(h, w3_ref[...]) + _dot(e, e_ref[...])
            acc = h if acc is None else jnp.maximum(acc, h)
        o_ref[...] = r_ref[...] + _dot(acc, wp_ref[...])

    return pl.pallas_call(
        body,
        grid=(N // R,),
        in_specs=[_edge_bs(R), _row_bs(R, D), _full_bs(w1.shape), _full_bs(w2.shape),
                  _full_bs(w3.shape), _full_bs(esel.shape), _full_bs(wproj.shape),
                  _row_bs(R, D)],
        out_specs=_row_bs(R, D),
        out_shape=jax.ShapeDtypeStruct((N, D), jnp.float32),
    )(tg, center, w1, w2, w3, esel, wproj, res)


def _tc_seg_mean(h0, R, seg_blocks):
    N, d = h0.shape

    def body(h_ref, o_ref):
        i = pl.program_id(0)

        @pl.when(i == 0)
        def _init():
            o_ref[...] = jnp.zeros_like(o_ref)

        s = jnp.sum(h_ref[...], axis=0, keepdims=True) * (1.0 / 25000.0)
        seg = i // seg_blocks
        rows = lax.broadcasted_iota(jnp.int32, (8, 1), 0)
        o_ref[...] += jnp.where(rows == seg, s, 0.0)

    return pl.pallas_call(
        body,
        grid=(N // R,),
        in_specs=[_row_bs(R, d)],
        out_specs=_full_bs((8, d)),
        out_shape=jax.ShapeDtypeStruct((8, d), jnp.float32),
    )(h0)


def _tc_dcd_head(mean8, fw1, fw2, f1w1, f1w2):
    d = mean8.shape[1]

    def body(m_ref, a1_ref, a2_ref, b1_ref, b2_ref, om_ref, ow_ref):
        m = m_ref[...]
        om_ref[...] = _gelu(_dot(_gelu(_dot(m, a1_ref[...])), a2_ref[...]))
        ow_ref[...] = jax.nn.sigmoid(_gelu(_dot(_gelu(_dot(m, b1_ref[...])), b2_ref[...])))

    return pl.pallas_call(
        body,
        grid=(1,),
        in_specs=[_full_bs(mean8.shape), _full_bs(fw1.shape), _full_bs(fw2.shape),
                  _full_bs(f1w1.shape), _full_bs(f1w2.shape)],
        out_specs=(_full_bs((8, d)), _full_bs((8, d))),
        out_shape=(jax.ShapeDtypeStruct((8, d), jnp.float32),
                   jax.ShapeDtypeStruct((8, d), jnp.float32)),
    )(mean8, fw1, fw2, f1w1, f1w2)


def _tc_final(h0, h1g, om, ow, cm, R, seg_blocks):
    N, d = h0.shape

    def body(h0_ref, h1_ref, om_ref, ow_ref, cm_ref, o_ref):
        seg = pl.program_id(0) // seg_blocks
        rows = lax.broadcasted_iota(jnp.int32, (8, 1), 0)
        msk = rows == seg
        om_s = jnp.sum(jnp.where(msk, om_ref[...], 0.0), axis=0, keepdims=True)
        ow_s = jnp.sum(jnp.where(msk, ow_ref[...], 0.0), axis=0, keepdims=True)
        h0v = h0_ref[...]
        o_ref[...] = (ow_s * 0.5 + 0.75) * h0v * cm_ref[...] + om_s * cm_ref[...] + h1_ref[...]

    return pl.pallas_call(
        body,
        grid=(N // R,),
        in_specs=[_row_bs(R, d), _row_bs(R, d), _full_bs((8, d)), _full_bs((8, d)),
                  _full_bs((1, d))],
        out_specs=_row_bs(R, d),
        out_shape=jax.ShapeDtypeStruct((N, d), jnp.float32),
    )(h0, h1g, om, ow, cm)


def _bns(g):
    return g * _S


def _emb(w, shape, r0=0, c0=0):
    out = jnp.zeros(shape, jnp.float32)
    return out.at[r0:r0 + w.shape[0], c0:c0 + w.shape[1]].set(w)


def _padv(g, n=D):
    return jnp.concatenate([g, jnp.zeros((n - g.shape[0],), jnp.float32)])


def _embed3_w(p):
    return p["w1"] * _bns(p["g1"])[None, :], p["w2"] * _bns(p["g2"])[None, :], p["w3"]


def _block_apply(f, knn_flat, p, Ng, R, Rg, gchunk):
    N = f.shape[0]
    m = p["mlp"]
    f = _tc_mlp_res(f, _emb(m["w1"], (D, m["w1"].shape[1])), m["b1"],
                    _emb(m["w2"] * _bns(m["g"])[None, :], (m["w2"].shape[0], D)), R)
    for i in range(4):
        lw = p["lfps"][i]["w"]
        y = _tc_matmul(f, _emb(lw, (D, D)), R)
        g3 = _gather_rows(y, knn_flat, gchunk).reshape(KNN, Ng, D)
        f = _tc_lfp_max(g3, y, f, _padv(_bns(p["lfps"][i]["g"])), Rg)
        if i % 2 == 1:
            m = p["mlps"][i // 2]
            f = _tc_mlp_res(f, _emb(m["w1"], (D, m["w1"].shape[1])), m["b1"],
                            _emb(m["w2"] * _bns(m["g"])[None, :], (m["w2"].shape[0], D)), R)
    return f


def kernel(x, xyz, knn0, knn1, ids1, back_nn1, params):
    p0, p1 = params["s0"], params["s1"]
    f32 = jnp.float32

    knn0_flat = jnp.pad(knn0.astype(jnp.int32).T, ((0, 0), (0, N0P - N0))).reshape(-1)
    knn1_flat = jnp.pad(knn1.astype(jnp.int32).T, ((0, 0), (0, N1P - N1))).reshape(-1)
    ids1_pad = jnp.pad(ids1.astype(jnp.int32), (0, N1P - N1))
    back_pad = jnp.pad(back_nn1.astype(jnp.int32), (0, N0BP - N0))

    xup = jnp.concatenate([xyz, x, jnp.zeros((N0, D - 7), f32)], axis=1)
    c0 = jnp.concatenate([xyz, jnp.zeros((N0, D - 3), f32)], axis=1)

    w1x, w2x, w3x = _embed3_w(p0["xemb"])
    t0 = _tc_embed3(x, w1x, w2x, _emb(w3x, (w3x.shape[0], D), c0=16), xup, 1000)

    w1n, w2n, w3n = _embed3_w(p0["nbr"])
    esel0 = _emb(jnp.eye(64, dtype=f32), (D, D), r0=16)
    tg0 = _gather_rows(t0, knn0_flat, 640).reshape(KNN, N0P, D)
    f0 = _tc_edge0(tg0, c0, _emb(w1n, (D, w1n.shape[1])), w2n,
                   _emb(w3n, (w3n.shape[0], D)), esel0, _padv(_bns(p0["nbr_bn_g"])),
                   N0, 400)
    f0 = _block_apply(f0, knn0_flat, p0["blk"], N0P, 1000, 400, 640)

    a = _tc_matmul(f0, _emb(p1["skip_w"] * _bns(p1["skip_g"])[None, :], (D, D)), 1000)
    y5 = _tc_matmul(f0, _emb(p1["lfp_w"], (D, D)), 1000)
    g5 = _gather_rows(y5, knn0_flat, 640).reshape(KNN, N0P, D)
    s_arr = _tc_lfp_max(g5, y5, a, _bns(p1["lfp_g"]), 400)

    f1 = _gather_rows(s_arr, ids1_pad, 400)
    xu1 = _gather_rows(c0, ids1_pad, 400)

    w1x1, w2x1, w3x1 = _embed3_w(p1["xemb"])
    t1c = _tc_embed3(f1, w1x1, w2x1, _emb(w3x1, (w3x1.shape[0], D), c0=16), xu1, 512)

    w1n1, w2n1, w3n1 = _embed3_w(p1["nbr"])
    esel1 = _emb(jnp.eye(32, dtype=f32), (D, 32), r0=16)
    tg1 = _gather_rows(t1c, knn1_flat, 640).reshape(KNN, N1P, D)
    f1 = _tc_edge1(tg1, xu1, _emb(w1n1, (D, w1n1.shape[1])), w2n1, w3n1, esel1,
                   p1["nbr_proj_w"] * _bns(p1["nbr_bn_g"])[None, :], f1, N1P, 512)
    f1 = _block_apply(f1, knn1_flat, p1["blk"], N1P, 512, 512, 640)

    t1 = _tc_matmul(f1, p1["post_w"] * _bns(p1["post_bn_g"])[:, None], 512)
    h1g = _gather_rows(t1, back_pad, 800)
    h0 = _tc_matmul(f0, _emb(p0["post_w"] * _bns(p0["post_bn_g"])[:, None], (D, D)), 1000)

    mean8 = _tc_seg_mean(h0, 1000, 25)
    fc, fc1 = p0["dcd"]["fc"], p0["dcd"]["fc1"]
    om, ow = _tc_dcd_head(mean8, fc["w1"] * _bns(fc["g1"])[None, :], fc["w2"],
                          fc1["w1"] * _bns(fc1["g1"])[None, :], fc1["w2"])
    return _tc_final(h0, h1g, om, ow, params["channel_matric"], 1000, 25)
